# baseline (speedup 1.0000x reference)
import functools

import jax
import jax.numpy as jnp
from jax import lax
from jax.experimental import pallas as pl
from jax.experimental.pallas import tpu as pltpu
from jax.experimental.pallas import tpu_sc as plsc

CHUNK = 128
LANES = 128


def _make_transpose(vocab: int, dim: int):
    info = plsc.get_sparse_core_info()
    nc, ns = info.num_cores, info.num_subcores
    nw = nc * ns

    n_full = vocab // LANES
    tail_rows = (vocab - n_full * LANES) * dim // 128
    rows_out = vocab * dim // 128

    mesh = plsc.VectorSubcoreMesh(core_axis_name="c", subcore_axis_name="s")
    nbuf = 4
    per_w = -(-n_full // nw)
    n_outer = -(-per_w // nbuf)

    @functools.partial(
        pl.kernel,
        mesh=mesh,
        out_type=jax.ShapeDtypeStruct((rows_out, 128), jnp.float32),
        scratch_types=[
            pltpu.VMEM((nbuf, dim, LANES), jnp.float32),
            pltpu.VMEM((nbuf, LANES * dim // 128, 128), jnp.float32),
            pltpu.SemaphoreType.DMA((nbuf,)),
            pltpu.SemaphoreType.DMA((nbuf,)),
        ],
        compiler_params=pltpu.CompilerParams(
            use_tc_tiling_on_sc=True, needs_layout_passes=False
        ),
    )
    def transpose(tt_hbm, tail_hbm, out_hbm, blk_v, trm_v, sem_in, sem_out):
        wid = lax.axis_index("s") * nc + lax.axis_index("c")

        @pl.when(wid == 0)
        def _():
            pltpu.sync_copy(tail_hbm, out_hbm.at[pl.ds(n_full * LANES * dim // 128, tail_rows)])

        def col0(m):
            j = wid + m * nw
            return j * LANES, j

        def start_in(m, b):
            c0, j = col0(m)

            @pl.when(j < n_full)
            def _():
                pltpu.make_async_copy(
                    tt_hbm.at[:, pl.ds(c0, LANES)], blk_v.at[b], sem_in.at[b]
                ).start()

        def wait_in(m, b):
            c0, j = col0(m)

            @pl.when(j < n_full)
            def _():
                pltpu.make_async_copy(
                    tt_hbm.at[:, pl.ds(c0, LANES)], blk_v.at[b], sem_in.at[b]
                ).wait()

        def out_copy(m, b):
            _, j = col0(m)
            rpb = LANES * dim // 128
            return pltpu.make_async_copy(
                trm_v.at[b],
                out_hbm.at[pl.ds(j * rpb, rpb)],
                sem_out.at[b],
            )

        def transpose_block(b):
            def body(i, carry):
                for g in range(dim // 16):
                    cids = lax.iota(jnp.int32, 16) + g * 16
                    bb = jnp.zeros((16,), jnp.int32) + b
                    left = plsc.load_gather(
                        blk_v, [bb, cids, jnp.zeros((16,), jnp.int32) + 2 * i]
                    )
                    right = plsc.load_gather(
                        blk_v, [bb, cids, jnp.zeros((16,), jnp.int32) + 2 * i + 1]
                    )
                    trm_v[b, i, pl.ds(g * 16, 16)] = left
                    trm_v[b, i, pl.ds(dim + g * 16, 16)] = right
                return carry

            lax.fori_loop(0, LANES // 2, body, 0)

        for b in range(nbuf):
            start_in(b, b)

        for b in range(nbuf):
            wait_in(b, b)

            @pl.when(col0(b)[1] < n_full)
            def _(b=b):
                transpose_block(b)
                out_copy(b, b).start()

            start_in(b + nbuf, b)

        def outer(t, carry):
            for b in range(nbuf):
                m = t * nbuf + b
                wait_in(m, b)

                @pl.when(col0(m)[1] < n_full)
                def _(m=m, b=b):
                    out_copy(m - nbuf, b).wait()
                    transpose_block(b)
                    out_copy(m, b).start()

                start_in(m + nbuf, b)
            return carry

        lax.fori_loop(1, n_outer, outer, 0)

        for b in range(nbuf):
            last_m = (n_outer - 1) * nbuf + b

            @pl.when(col0(last_m)[1] < n_full)
            def _(last_m=last_m, b=b):
                out_copy(last_m, b).wait()

            @pl.when(
                jnp.logical_and(
                    col0(last_m)[1] >= n_full,
                    col0(last_m - nbuf)[1] < n_full,
                )
            )
            def _(last_m=last_m, b=b):
                out_copy(last_m - nbuf, b).wait()

    return transpose


def _make_gather(n_chunks: int, vocab: int, dim: int):
    info = plsc.get_sparse_core_info()
    nc, ns = info.num_cores, info.num_subcores
    nw = nc * ns
    per_w = n_chunks // nw

    mesh = plsc.VectorSubcoreMesh(core_axis_name="c", subcore_axis_name="s")

    nbuf = 10
    assert per_w % nbuf == 0
    n_outer = per_w // nbuf

    @functools.partial(
        pl.kernel,
        mesh=mesh,
        out_type=jax.ShapeDtypeStruct((n_chunks * CHUNK, dim), jnp.float32),
        scratch_types=[
            pltpu.VMEM((per_w * CHUNK,), jnp.int32),
            pltpu.VMEM((nbuf, CHUNK, dim), jnp.float32),
            pltpu.SemaphoreType.DMA((nbuf,)),
            pltpu.SemaphoreType.DMA((nbuf,)),
        ],
        compiler_params=pltpu.CompilerParams(use_tc_tiling_on_sc=False),
    )
    def gather(idx_hbm, table_hbm, out_hbm, idx_v, rows_v, sem_in, sem_out):
        wid = lax.axis_index("s") * nc + lax.axis_index("c")
        base_chunk = wid * per_w
        pltpu.sync_copy(idx_hbm.at[pl.ds(base_chunk * CHUNK, per_w * CHUNK)], idx_v)

        def gather_chunk(c, b):
            return pltpu.make_async_copy(
                table_hbm.at[idx_v.at[pl.ds(c * CHUNK, CHUNK)]],
                rows_v.at[b],
                sem_in.at[b],
            )

        def write_chunk(c, b):
            return pltpu.make_async_copy(
                rows_v.at[b],
                out_hbm.at[pl.ds((base_chunk + c) * CHUNK, CHUNK)],
                sem_out.at[b],
            )

        for b in range(nbuf):
            gather_chunk(b, b).start()

        def outer(g, carry):
            for b in range(nbuf):
                c = g * nbuf + b
                gather_chunk(c, b).wait()
                write_chunk(c, b).start()
                write_chunk(c, b).wait()
                gather_chunk(c + nbuf, b).start()
            return carry

        lax.fori_loop(0, n_outer - 1, outer, 0)

        for b in range(nbuf):
            c = (n_outer - 1) * nbuf + b
            gather_chunk(c, b).wait()
            write_chunk(c, b).start()
        for b in range(nbuf):
            c = (n_outer - 1) * nbuf + b
            write_chunk(c, b).wait()

    return gather


def kernel(words, table):
    b, s = words.shape
    vocab, dim = table.shape
    n = b * s
    assert n % CHUNK == 0
    n_chunks = n // CHUNK
    n_full = vocab // LANES
    tail_pack = table[n_full * LANES :].reshape(-1, 128)
    t_pack = _make_transpose(vocab, dim)(table.T, tail_pack)
    t_lin = t_pack.reshape(vocab, dim)
    idx_flat = words.reshape(n).astype(jnp.int32)
    out = _make_gather(n_chunks, vocab, dim)(idx_flat, t_lin)
    return out.reshape(b, s, dim)


# speedup vs baseline: 1.3568x; 1.3568x over previous
import functools

import jax
import jax.numpy as jnp
from jax import lax
from jax.experimental import pallas as pl
from jax.experimental.pallas import tpu as pltpu
from jax.experimental.pallas import tpu_sc as plsc

CHUNK = 128
LANES = 128


def _make_transpose(vocab: int, dim: int):
    info = plsc.get_sparse_core_info()
    nc, ns = info.num_cores, info.num_subcores
    nw = nc * ns

    n_full = vocab // LANES
    tail_rows = (vocab - n_full * LANES) * dim // 128
    rows_out = vocab * dim // 128

    mesh = plsc.VectorSubcoreMesh(core_axis_name="c", subcore_axis_name="s")
    nbuf = 4
    per_w = -(-n_full // nw)
    n_outer = -(-per_w // nbuf)

    @functools.partial(
        pl.kernel,
        mesh=mesh,
        out_type=jax.ShapeDtypeStruct((rows_out, 128), jnp.float32),
        scratch_types=[
            pltpu.VMEM((nbuf, dim, LANES), jnp.float32),
            pltpu.VMEM((nbuf, LANES * dim // 128, 128), jnp.float32),
            pltpu.SemaphoreType.DMA((nbuf,)),
            pltpu.SemaphoreType.DMA((nbuf,)),
        ],
        compiler_params=pltpu.CompilerParams(
            use_tc_tiling_on_sc=True, needs_layout_passes=False
        ),
    )
    def transpose(tt_hbm, tail_hbm, out_hbm, blk_v, trm_v, sem_in, sem_out):
        wid = lax.axis_index("s") * nc + lax.axis_index("c")

        @pl.when(wid == 0)
        def _():
            pltpu.sync_copy(tail_hbm, out_hbm.at[pl.ds(n_full * LANES * dim // 128, tail_rows)])

        def col0(m):
            j = wid + m * nw
            return j * LANES, j

        def start_in(m, b):
            c0, j = col0(m)

            @pl.when(j < n_full)
            def _():
                pltpu.make_async_copy(
                    tt_hbm.at[:, pl.ds(c0, LANES)], blk_v.at[b], sem_in.at[b]
                ).start()

        def wait_in(m, b):
            c0, j = col0(m)

            @pl.when(j < n_full)
            def _():
                pltpu.make_async_copy(
                    tt_hbm.at[:, pl.ds(c0, LANES)], blk_v.at[b], sem_in.at[b]
                ).wait()

        def out_copy(m, b):
            _, j = col0(m)
            rpb = LANES * dim // 128
            return pltpu.make_async_copy(
                trm_v.at[b],
                out_hbm.at[pl.ds(j * rpb, rpb)],
                sem_out.at[b],
            )

        cids = [lax.iota(jnp.int32, 16) + g * 16 for g in range(dim // 16)]
        zeros16 = jnp.zeros((16,), jnp.int32)

        def transpose_block(b):
            bb = zeros16 + b

            @plsc.parallel_loop(0, LANES // 2, unroll=4)
            def body(i):
                for g in range(dim // 16):
                    left = plsc.load_gather(blk_v, [bb, cids[g], zeros16 + 2 * i])
                    right = plsc.load_gather(
                        blk_v, [bb, cids[g], zeros16 + 2 * i + 1]
                    )
                    trm_v[b, i, pl.ds(g * 16, 16)] = left
                    trm_v[b, i, pl.ds(dim + g * 16, 16)] = right

        for b in range(nbuf):
            start_in(b, b)

        for b in range(nbuf):
            wait_in(b, b)

            @pl.when(col0(b)[1] < n_full)
            def _(b=b):
                transpose_block(b)
                out_copy(b, b).start()

            start_in(b + nbuf, b)

        def outer(t, carry):
            for b in range(nbuf):
                m = t * nbuf + b
                wait_in(m, b)

                @pl.when(col0(m)[1] < n_full)
                def _(m=m, b=b):
                    out_copy(m - nbuf, b).wait()
                    transpose_block(b)
                    out_copy(m, b).start()

                start_in(m + nbuf, b)
            return carry

        lax.fori_loop(1, n_outer, outer, 0)

        for b in range(nbuf):
            last_m = (n_outer - 1) * nbuf + b

            @pl.when(col0(last_m)[1] < n_full)
            def _(last_m=last_m, b=b):
                out_copy(last_m, b).wait()

            @pl.when(
                jnp.logical_and(
                    col0(last_m)[1] >= n_full,
                    col0(last_m - nbuf)[1] < n_full,
                )
            )
            def _(last_m=last_m, b=b):
                out_copy(last_m - nbuf, b).wait()

    return transpose


def _make_gather(n_chunks: int, vocab: int, dim: int):
    info = plsc.get_sparse_core_info()
    nc, ns = info.num_cores, info.num_subcores
    nw = nc * ns
    per_w = n_chunks // nw

    mesh = plsc.VectorSubcoreMesh(core_axis_name="c", subcore_axis_name="s")

    nbuf = 10
    assert per_w % nbuf == 0
    n_outer = per_w // nbuf

    @functools.partial(
        pl.kernel,
        mesh=mesh,
        out_type=jax.ShapeDtypeStruct((n_chunks * CHUNK, dim), jnp.float32),
        scratch_types=[
            pltpu.VMEM((per_w * CHUNK,), jnp.int32),
            pltpu.VMEM((nbuf, CHUNK, dim), jnp.float32),
            pltpu.SemaphoreType.DMA((nbuf,)),
            pltpu.SemaphoreType.DMA((nbuf,)),
        ],
        compiler_params=pltpu.CompilerParams(use_tc_tiling_on_sc=False),
    )
    def gather(idx_hbm, table_hbm, out_hbm, idx_v, rows_v, sem_in, sem_out):
        wid = lax.axis_index("s") * nc + lax.axis_index("c")
        base_chunk = wid * per_w
        pltpu.sync_copy(idx_hbm.at[pl.ds(base_chunk * CHUNK, per_w * CHUNK)], idx_v)

        def gather_chunk(c, b):
            return pltpu.make_async_copy(
                table_hbm.at[idx_v.at[pl.ds(c * CHUNK, CHUNK)]],
                rows_v.at[b],
                sem_in.at[b],
            )

        def write_chunk(c, b):
            return pltpu.make_async_copy(
                rows_v.at[b],
                out_hbm.at[pl.ds((base_chunk + c) * CHUNK, CHUNK)],
                sem_out.at[b],
            )

        for b in range(nbuf):
            gather_chunk(b, b).start()

        def outer(g, carry):
            for b in range(nbuf):
                c = g * nbuf + b
                gather_chunk(c, b).wait()
                write_chunk(c, b).start()
                write_chunk(c, b).wait()
                gather_chunk(c + nbuf, b).start()
            return carry

        lax.fori_loop(0, n_outer - 1, outer, 0)

        for b in range(nbuf):
            c = (n_outer - 1) * nbuf + b
            gather_chunk(c, b).wait()
            write_chunk(c, b).start()
        for b in range(nbuf):
            c = (n_outer - 1) * nbuf + b
            write_chunk(c, b).wait()

    return gather


def kernel(words, table):
    b, s = words.shape
    vocab, dim = table.shape
    n = b * s
    assert n % CHUNK == 0
    n_chunks = n // CHUNK
    n_full = vocab // LANES
    tail_pack = table[n_full * LANES :].reshape(-1, 128)
    t_pack = _make_transpose(vocab, dim)(table.T, tail_pack)
    t_lin = t_pack.reshape(vocab, dim)
    idx_flat = words.reshape(n).astype(jnp.int32)
    out = _make_gather(n_chunks, vocab, dim)(idx_flat, t_lin)
    return out.reshape(b, s, dim)


# speedup vs baseline: 1.7091x; 1.2596x over previous
import functools

import jax
import jax.numpy as jnp
from jax import lax
from jax.experimental import pallas as pl
from jax.experimental.pallas import tpu as pltpu
from jax.experimental.pallas import tpu_sc as plsc

CHUNK = 128


def _make_gather(n_chunks: int, vocab: int, dim: int):
    info = plsc.get_sparse_core_info()
    nc, ns = info.num_cores, info.num_subcores
    nw = nc * ns
    per_w = n_chunks // nw

    mesh = plsc.VectorSubcoreMesh(core_axis_name="c", subcore_axis_name="s")

    nbuf = 10
    assert per_w % nbuf == 0
    n_outer = per_w // nbuf

    @functools.partial(
        pl.kernel,
        mesh=mesh,
        out_type=jax.ShapeDtypeStruct((n_chunks * CHUNK, dim), jnp.float32),
        scratch_types=[
            pltpu.VMEM((per_w * CHUNK,), jnp.int32),
            pltpu.VMEM((nbuf, CHUNK, dim), jnp.float32),
            pltpu.SemaphoreType.DMA((nbuf,)),
            pltpu.SemaphoreType.DMA((nbuf,)),
        ],
        compiler_params=pltpu.CompilerParams(use_tc_tiling_on_sc=False),
    )
    def gather(idx_hbm, table_hbm, out_hbm, idx_v, rows_v, sem_in, sem_out):
        wid = lax.axis_index("s") * nc + lax.axis_index("c")
        base_chunk = wid * per_w
        pltpu.sync_copy(idx_hbm.at[pl.ds(base_chunk * CHUNK, per_w * CHUNK)], idx_v)

        def gather_chunk(c, b):
            return pltpu.make_async_copy(
                table_hbm.at[idx_v.at[pl.ds(c * CHUNK, CHUNK)]],
                rows_v.at[b],
                sem_in.at[b],
            )

        def write_chunk(c, b):
            return pltpu.make_async_copy(
                rows_v.at[b],
                out_hbm.at[pl.ds((base_chunk + c) * CHUNK, CHUNK)],
                sem_out.at[b],
            )

        for b in range(nbuf):
            gather_chunk(b, b).start()

        def outer(g, carry):
            for b in range(nbuf):
                c = g * nbuf + b
                gather_chunk(c, b).wait()
                write_chunk(c, b).start()
                write_chunk(c, b).wait()
                gather_chunk(c + nbuf, b).start()
            return carry

        lax.fori_loop(0, n_outer - 1, outer, 0)

        for b in range(nbuf):
            c = (n_outer - 1) * nbuf + b
            gather_chunk(c, b).wait()
            write_chunk(c, b).start()
        for b in range(nbuf):
            c = (n_outer - 1) * nbuf + b
            write_chunk(c, b).wait()

    return gather


def kernel(words, table):
    b, s = words.shape
    vocab, dim = table.shape
    n = b * s
    assert n % CHUNK == 0
    n_chunks = n // CHUNK
    idx_flat = words.reshape(n).astype(jnp.int32)
    out = _make_gather(n_chunks, vocab, dim)(idx_flat, table)
    return out.reshape(b, s, dim)
